# Initial kernel scaffold; baseline (speedup 1.0000x reference)
import jax
import jax.numpy as jnp
from jax.experimental import pallas as pl


def kernel(x_ids, lengths, emb, W, b):
    raise NotImplementedError("write your pallas kernel here")


# speedup vs baseline: 14.9835x; 14.9835x over previous
import functools

import jax
import jax.numpy as jnp
from jax.experimental import pallas as pl
from jax.experimental.pallas import tpu as pltpu

B, L, C = 8, 512, 52
V, E, D = 128, 64, 256
NPAIR = C // 2


def _table_body(emb_ref, w_ref, t_ref):
    t_ref[0] = jax.lax.dot(
        emb_ref[...], w_ref[0], preferred_element_type=jnp.float32
    ).astype(jnp.bfloat16)


def _build_table(emb, w3):
    return pl.pallas_call(
        _table_body,
        grid=(C,),
        in_specs=[
            pl.BlockSpec((V, E), lambda c: (0, 0)),
            pl.BlockSpec((1, E, D), lambda c: (c, 0, 0)),
        ],
        out_specs=pl.BlockSpec((1, V, D), lambda c: (c, 0, 0)),
        out_shape=jax.ShapeDtypeStruct((C, V, D), jnp.bfloat16),
    )(emb, w3)


def _fused_body(order_ref, slen_ref, ids_ref, t2_ref, b_ref, y_ref):
    m = pl.program_id(0)
    sl = slen_ref[m]
    ids = ids_ref[0]
    pos = jax.lax.broadcasted_iota(jnp.int32, (L, 1), 0)
    valid = pos < sl
    col = jax.lax.broadcasted_iota(jnp.int32, (L, 2 * V), 1)
    in_lo = col < V
    acc = jnp.full((L, D), 0.0, dtype=jnp.float32)
    for cc in range(NPAIR):
        id0 = jnp.where(valid, ids[:, 2 * cc : 2 * cc + 1], 0)
        id1 = jnp.where(valid, ids[:, 2 * cc + 1 : 2 * cc + 2], 0)
        sel = jnp.where(in_lo, id0, id1 + V)
        a2 = (col == sel).astype(jnp.bfloat16)
        acc += jax.lax.dot(a2, t2_ref[cc], preferred_element_type=jnp.float32)
    y_ref[0] = jax.nn.relu(acc + b_ref[...])


def _fused(x_ids, order, slen, t2, b2):
    grid_spec = pltpu.PrefetchScalarGridSpec(
        num_scalar_prefetch=2,
        grid=(B,),
        in_specs=[
            pl.BlockSpec((1, L, C), lambda m, order_ref, slen_ref: (order_ref[m], 0, 0)),
            pl.BlockSpec((NPAIR, 2 * V, D), lambda m, *_: (0, 0, 0)),
            pl.BlockSpec((1, D), lambda m, *_: (0, 0)),
        ],
        out_specs=pl.BlockSpec((1, L, D), lambda m, *_: (m, 0, 0)),
    )
    return pl.pallas_call(
        _fused_body,
        grid_spec=grid_spec,
        out_shape=jax.ShapeDtypeStruct((B, L, D), jnp.float32),
    )(order, slen, x_ids, t2, b2)


@functools.partial(jax.jit, static_argnames=())
def kernel(x_ids, lengths, emb, W, b):
    order = jnp.argsort(-lengths, stable=True).astype(jnp.int32)
    sorted_len = lengths[order]
    reversed_indices = jnp.argsort(order, stable=True)

    t = _build_table(emb, W.reshape(C, E, D))
    t2 = t.reshape(NPAIR, 2 * V, D)
    y = _fused(
        x_ids.astype(jnp.int32),
        order,
        sorted_len.astype(jnp.int32),
        t2,
        b.reshape(1, D),
    )
    return (y, sorted_len, reversed_indices)
